# Initial kernel scaffold; baseline (speedup 1.0000x reference)
import jax
import jax.numpy as jnp
from jax.experimental import pallas as pl


def kernel(src_coords, tgt_coords):
    raise NotImplementedError("write your pallas kernel here")


# speedup vs baseline: 66.0383x; 66.0383x over previous
import functools

import jax
import jax.numpy as jnp
from jax import lax
from jax.experimental import pallas as pl
from jax.experimental.pallas import tpu as pltpu
from jax.experimental.pallas import tpu_sc as plsc

N = 8192
SB = 256
TB = 1024
BIG = 2 ** 30
RADIUS = 0.001
EPS = 1e-07
PD_EPS = 1e-06
R2 = RADIUS * RADIUS

NUM_SUBCORES = 32
NPER = N // NUM_SUBCORES


def _topk2_body(src_ref, tgt_ref, out_ref, m1, i1, m2, i2):
    jb = pl.program_id(1)
    nj = pl.num_programs(1)

    @pl.when(jb == 0)
    def _init():
        m1[...] = jnp.full((SB, TB), jnp.inf, jnp.float32)
        m2[...] = jnp.full((SB, TB), jnp.inf, jnp.float32)
        i1[...] = jnp.zeros((SB, TB), jnp.int32)
        i2[...] = jnp.zeros((SB, TB), jnp.int32)

    s = src_ref[...]
    t = tgt_ref[...]
    dx = s[:, 0:1] - t[0:1, :]
    dy = s[:, 1:2] - t[1:2, :]
    dz = s[:, 2:3] - t[2:3, :]
    d2 = dx * dx + dy * dy + dz * dz

    jj = lax.broadcasted_iota(jnp.int32, (1, TB), 1) + jb * TB
    jj = jnp.broadcast_to(jj, (SB, TB))

    vm1 = m1[...]
    vm2 = m2[...]
    vi1 = i1[...]
    vi2 = i2[...]
    lt1 = d2 < vm1
    lt2 = d2 < vm2
    m2[...] = jnp.where(lt1, vm1, jnp.where(lt2, d2, vm2))
    i2[...] = jnp.where(lt1, vi1, jnp.where(lt2, jj, vi2))
    m1[...] = jnp.where(lt1, d2, vm1)
    i1[...] = jnp.where(lt1, jj, vi1)

    @pl.when(jb == nj - 1)
    def _finalize():
        fm1 = m1[...]
        fi1 = i1[...]
        fm2 = m2[...]
        fi2 = i2[...]
        rowm1 = jnp.min(fm1, axis=1, keepdims=True)
        ismin = fm1 == rowm1
        ia = jnp.min(jnp.where(ismin, fi1, BIG), axis=1, keepdims=True)
        chosen = ismin & (fi1 == ia)
        m1a = jnp.where(chosen, fm2, fm1)
        i1a = jnp.where(chosen, fi2, fi1)
        rowm2 = jnp.min(m1a, axis=1, keepdims=True)
        idx2 = jnp.min(jnp.where(m1a == rowm2, i1a, BIG), axis=1, keepdims=True)
        out_ref[...] = idx2


def _second_nn_idx(src_pad, tgt_t):
    return pl.pallas_call(
        _topk2_body,
        grid=(N // SB, N // TB),
        in_specs=[
            pl.BlockSpec((SB, 8), lambda ib, jb: (ib, 0)),
            pl.BlockSpec((8, TB), lambda ib, jb: (0, jb)),
        ],
        out_specs=pl.BlockSpec((SB, 1), lambda ib, jb: (ib, 0)),
        out_shape=jax.ShapeDtypeStruct((N, 1), jnp.int32),
        scratch_shapes=[
            pltpu.VMEM((SB, TB), jnp.float32),
            pltpu.VMEM((SB, TB), jnp.int32),
            pltpu.VMEM((SB, TB), jnp.float32),
            pltpu.VMEM((SB, TB), jnp.int32),
        ],
        compiler_params=pltpu.CompilerParams(
            dimension_semantics=("arbitrary", "arbitrary")),
    )(src_pad, tgt_t)


def _match_counts(idx, sx, sy, sz, tx, ty, tz):
    mesh = plsc.VectorSubcoreMesh(core_axis_name="c", subcore_axis_name="s")

    @functools.partial(
        pl.kernel,
        mesh=mesh,
        out_type=jax.ShapeDtypeStruct((NUM_SUBCORES, 16), jnp.float32),
        scratch_types=[
            pltpu.VMEM((NPER,), jnp.int32),
            pltpu.VMEM((NPER,), jnp.float32),
            pltpu.VMEM((NPER,), jnp.float32),
            pltpu.VMEM((NPER,), jnp.float32),
            pltpu.VMEM((NPER,), jnp.float32),
            pltpu.VMEM((NPER,), jnp.float32),
            pltpu.VMEM((NPER,), jnp.float32),
            pltpu.VMEM((16,), jnp.float32),
            pltpu.SemaphoreType.DMA,
        ],
    )
    def k(idx_hbm, sx_hbm, sy_hbm, sz_hbm, tx_hbm, ty_hbm, tz_hbm, out_hbm,
          idx_v, gx_v, gy_v, gz_v, sx_v, sy_v, sz_v, cnt_v, sem):
        wid = lax.axis_index("c") * 16 + lax.axis_index("s")
        base = wid * NPER
        pltpu.sync_copy(idx_hbm.at[pl.ds(base, NPER)], idx_v)
        pltpu.async_copy(tx_hbm.at[idx_v], gx_v, sem).wait()
        pltpu.async_copy(ty_hbm.at[idx_v], gy_v, sem).wait()
        pltpu.async_copy(tz_hbm.at[idx_v], gz_v, sem).wait()
        pltpu.sync_copy(sx_hbm.at[pl.ds(base, NPER)], sx_v)
        pltpu.sync_copy(sy_hbm.at[pl.ds(base, NPER)], sy_v)
        pltpu.sync_copy(sz_hbm.at[pl.ds(base, NPER)], sz_v)
        cnt = jnp.zeros((16,), jnp.float32)
        one = jnp.full((16,), 1.0, jnp.float32)
        zero = jnp.zeros((16,), jnp.float32)
        for i in range(NPER // 16):
            sl = pl.ds(i * 16, 16)
            ax = jnp.abs(sx_v[sl] - gx_v[sl] + PD_EPS)
            ay = jnp.abs(sy_v[sl] - gy_v[sl] + PD_EPS)
            az = jnp.abs(sz_v[sl] - gz_v[sl] + PD_EPS)
            s2 = ax * ax + ay * ay + az * az
            cnt = cnt + jnp.where(s2 < R2, one, zero)
        cnt_v[...] = cnt
        pltpu.sync_copy(cnt_v, out_hbm.at[wid])

    return k(idx, sx, sy, sz, tx, ty, tz)


def kernel(src_coords, tgt_coords):
    src_pad = jnp.pad(src_coords, ((0, 0), (0, 5)))
    tgt_t = jnp.pad(tgt_coords.T, ((0, 5), (0, 0)))
    idx2 = _second_nn_idx(src_pad, tgt_t)[:, 0]

    counts = _match_counts(
        idx2,
        src_coords[:, 0], src_coords[:, 1], src_coords[:, 2],
        tgt_coords[:, 0], tgt_coords[:, 1], tgt_coords[:, 2],
    )
    n1 = jnp.sum(counts)
    n0 = jnp.float32(N) - n1
    lse = jnp.log(n0 * jnp.exp(jnp.float32(EPS))
                  + n1 * jnp.exp(jnp.float32(1.0 + EPS)))
    return jax.nn.softplus(lse)
